# Initial kernel scaffold; baseline (speedup 1.0000x reference)
import jax
import jax.numpy as jnp
from jax.experimental import pallas as pl


def kernel(x, edge_index, edge_attr, W1l, b1, W1r, W2l, b2, W2r):
    raise NotImplementedError("write your pallas kernel here")


# speedup vs baseline: 5.5340x; 5.5340x over previous
import functools

import jax
import jax.numpy as jnp
from jax import lax
from jax.experimental import pallas as pl
from jax.experimental.pallas import tpu as pltpu
from jax.experimental.pallas import tpu_sc as plsc

N = 10000
E = 320000
D = 128
NC = 2
NS = 16
NW = NC * NS
E_TILE = E // NW
CH = 80
N_CH = E_TILE // CH
NP = 10240
ROWS_TILE = NP // NS
ZC = ROWS_TILE // CH


def _sc_segment_sum(with_deg):
  mesh = plsc.VectorSubcoreMesh(core_axis_name="c", subcore_axis_name="s",
                                num_cores=NC, num_subcores=NS)
  out_type = [jax.ShapeDtypeStruct((NC, NP, D), jnp.float32)]
  scratch = [
      pltpu.VMEM((CH,), jnp.int32),
      pltpu.VMEM((CH,), jnp.int32),
      pltpu.VMEM((CH, D), jnp.float32),
      pltpu.VMEM_SHARED((NP, D), jnp.float32),
      pltpu.SemaphoreType.DMA,
  ]
  if with_deg:
    out_type.append(jax.ShapeDtypeStruct((NC, NS, ROWS_TILE), jnp.float32))
    scratch += [
        pltpu.VMEM((NP,), jnp.float32),
        pltpu.VMEM((ROWS_TILE,), jnp.float32),
        pltpu.VMEM((ROWS_TILE,), jnp.float32),
        pltpu.VMEM_SHARED((NS, NP), jnp.float32),
    ]

  def body(x_hbm, src_hbm, dst_hbm, *refs):
    if with_deg:
      (out_hbm, deg_hbm, src_v, dst_v, rows_v, acc_sh, sem,
       deg_t, rowb_v, degf_v, dstage_sh) = refs
    else:
      out_hbm, src_v, dst_v, rows_v, acc_sh, sem = refs
    c = lax.axis_index("c")
    s = lax.axis_index("s")
    zvec = jnp.zeros((16,), jnp.float32)
    ones16 = jnp.full((16,), 1.0, jnp.float32)

    def zero_row(i, _):
      def zero_block(j, _):
        rows_v[i, pl.ds(j * 16, 16)] = zvec
        return 0
      return lax.fori_loop(0, D // 16, zero_block, 0)
    lax.fori_loop(0, CH, zero_row, 0)
    stripe0 = s * ROWS_TILE
    for q in range(ZC):
      pltpu.sync_copy(rows_v, acc_sh.at[pl.ds(stripe0 + q * CH, CH)])
    if with_deg:
      def zero_deg(i, _):
        deg_t[pl.ds(i * 16, 16)] = zvec
        return 0
      lax.fori_loop(0, NP // 16, zero_deg, 0)
    plsc.subcore_barrier()

    ebase = (c * NS + s) * E_TILE

    def step(g, _):
      off = pl.multiple_of(ebase + g * CH, 8)
      pltpu.sync_copy(src_hbm.at[pl.ds(off, CH)], src_v)
      pltpu.sync_copy(dst_hbm.at[pl.ds(off, CH)], dst_v)
      pltpu.async_copy(x_hbm.at[src_v], rows_v, sem).wait()
      pltpu.sync_copy(rows_v, acc_sh.at[dst_v], add=True)
      if with_deg:
        for k in range(CH // 16):
          idx = dst_v[pl.ds(k * 16, 16)]
          plsc.addupdate_scatter(deg_t, [idx], ones16)
      return 0

    lax.fori_loop(0, N_CH, step, 0)
    if with_deg:
      pltpu.sync_copy(deg_t, dstage_sh.at[s])
    plsc.subcore_barrier()

    pltpu.sync_copy(acc_sh.at[pl.ds(stripe0, ROWS_TILE)],
                    out_hbm.at[c, pl.ds(stripe0, ROWS_TILE)])
    if with_deg:
      def zf(i, _):
        degf_v[pl.ds(i * 16, 16)] = zvec
        return 0
      lax.fori_loop(0, ROWS_TILE // 16, zf, 0)
      for r in range(NS):
        pltpu.sync_copy(dstage_sh.at[r, pl.ds(stripe0, ROWS_TILE)], rowb_v)
        def acc_red(i, _):
          sl = pl.ds(i * 16, 16)
          degf_v[sl] = degf_v[sl] + rowb_v[sl]
          return 0
        lax.fori_loop(0, ROWS_TILE // 16, acc_red, 0)
      pltpu.sync_copy(degf_v, deg_hbm.at[c, s])

  return pl.kernel(
      body, out_type=out_type, mesh=mesh, scratch_types=scratch,
      compiler_params=pltpu.CompilerParams(needs_layout_passes=False))


_sc_pass_deg = _sc_segment_sum(with_deg=True)
_sc_pass = _sc_segment_sum(with_deg=False)


def _tc_layer_body(relu, sums_ref, degs_ref, h_ref, wl_ref, bl_ref, wr_ref,
                   o_ref):
  ssum = sums_ref[0] + sums_ref[1]
  deg = degs_ref[0] + degs_ref[1]
  agg = ssum / jnp.maximum(deg, 1.0)
  out = (jnp.dot(agg, wl_ref[...], preferred_element_type=jnp.float32)
         + bl_ref[...]
         + jnp.dot(h_ref[...], wr_ref[...], preferred_element_type=jnp.float32))
  nrm = jnp.sqrt(jnp.sum(out * out, axis=1, keepdims=True))
  out = out / jnp.maximum(nrm, 1e-12)
  if relu:
    out = jnp.maximum(out, 0.0)
  o_ref[...] = out


def _tc_layer(sums, degs, h, wl, bl, wr, relu, bn=1000):
  grid = N // bn
  return pl.pallas_call(
      functools.partial(_tc_layer_body, relu),
      grid=(grid,),
      in_specs=[
          pl.BlockSpec((NC, bn, D), lambda i: (0, i, 0)),
          pl.BlockSpec((NC, bn, 1), lambda i: (0, i, 0)),
          pl.BlockSpec((bn, D), lambda i: (i, 0)),
          pl.BlockSpec((D, D), lambda i: (0, 0)),
          pl.BlockSpec((1, D), lambda i: (0, 0)),
          pl.BlockSpec((D, D), lambda i: (0, 0)),
      ],
      out_specs=pl.BlockSpec((bn, D), lambda i: (i, 0)),
      out_shape=jax.ShapeDtypeStruct((N, D), jnp.float32),
  )(sums, degs, h, wl, bl, wr)


def kernel(x, edge_index, edge_attr, W1l, b1, W1r, W2l, b2, W2r):
  src = edge_index[0].astype(jnp.int32)
  dst = edge_index[1].astype(jnp.int32)
  b1r = b1.reshape(1, D)
  b2r = b2.reshape(1, D)

  sums1, degs = _sc_pass_deg(x, src, dst)
  degs3 = degs.reshape(NC, NP, 1)
  h1 = _tc_layer(sums1, degs3, x, W1l, b1r, W1r, relu=True)
  (sums2,) = _sc_pass(h1, src, dst)
  h2 = _tc_layer(sums2, degs3, h1, W2l, b2r, W2r, relu=False)
  return h2
